# Initial kernel scaffold; baseline (speedup 1.0000x reference)
import jax
import jax.numpy as jnp
from jax.experimental import pallas as pl


def kernel(padded_input, lengths):
    raise NotImplementedError("write your pallas kernel here")


# speedup vs baseline: 5.6389x; 5.6389x over previous
import numpy as np

import jax
import jax.numpy as jnp
from jax import lax
from jax.experimental import pallas as pl
from jax.experimental.pallas import tpu as pltpu
from jax.experimental.pallas import tpu_sc as plsc

B, T, D = 16, 4096, 128
NC, NS, L = 2, 16, 16
NW = NC * NS
DCH = D // L
NG = B * DCH
GPW = NG // NW

_SIGN = np.uint32(0x80000000)


def _tec_kernel(x_hbm, len_hbm, out_hbm, buf, hist, lenv, outv):
    wid = lax.axis_index("s") * NC + lax.axis_index("c")
    pltpu.sync_copy(len_hbm, lenv)
    lanes = lax.iota(jnp.int32, L)
    ones = jnp.ones((L,), jnp.int32)
    lens = lenv[...]

    for i in range(GPW):
        g = wid + NW * i
        b = g // DCH
        dc = g % DCH
        pltpu.sync_copy(x_hbm.at[b, :, pl.ds(dc * L, L)], buf)
        l = jnp.sum(jnp.where(lanes == b, lens, 0))
        k0 = (l - 1) // 2

        prefix = jnp.zeros((L,), jnp.uint32)
        kvec = jnp.broadcast_to(k0, (L,))
        for p in range(4):
            def zero_hist(j, c):
                hist[j] = jnp.zeros((L,), jnp.int32)
                return c
            lax.fori_loop(0, 256, zero_hist, 0)

            if p == 0:
                def body(t, pfx):
                    xi = plsc.bitcast(buf[t], jnp.uint32)
                    key = jnp.where(xi >> 31 == 1, ~xi, xi ^ _SIGN)
                    bkt = (key >> 24).astype(jnp.int32)
                    plsc.addupdate_scatter(hist, [bkt, lanes], ones)
                    return pfx
            else:
                hi_sh, lo_sh = 32 - 8 * p, 24 - 8 * p

                def body(t, pfx, hi_sh=hi_sh, lo_sh=lo_sh):
                    xi = plsc.bitcast(buf[t], jnp.uint32)
                    key = jnp.where(xi >> 31 == 1, ~xi, xi ^ _SIGN)
                    match = (key >> hi_sh) == pfx
                    bkt = ((key >> lo_sh) & 255).astype(jnp.int32)
                    plsc.addupdate_scatter(hist, [bkt, lanes],
                                           jnp.where(match, 1, 0))
                    return pfx
            lax.fori_loop(0, l, body, prefix)

            def scan_hist(j, c):
                cnt, sel, base, kv = c
                row = hist[j]
                ncnt = cnt + row
                below = ncnt <= kv
                sel = sel + jnp.where(below, 1, 0)
                base = base + jnp.where(below, row, 0)
                return ncnt, sel, base, kv
            init = (jnp.zeros((L,), jnp.int32), jnp.zeros((L,), jnp.int32),
                    jnp.zeros((L,), jnp.int32), kvec)
            _, sel, base, _ = lax.fori_loop(0, 256, scan_hist, init)
            prefix = (prefix << 8) | sel.astype(jnp.uint32)
            kvec = kvec - base

        bits = jnp.where(prefix >> 31 == 1, prefix ^ _SIGN, ~prefix)
        outv[...] = plsc.bitcast(bits, jnp.float32)
        pltpu.sync_copy(outv, out_hbm.at[b, pl.ds(dc * L, L)])


@jax.jit
def _sc_median(x, lengths_i32):
    mesh = plsc.VectorSubcoreMesh(core_axis_name="c", subcore_axis_name="s",
                                  num_cores=NC, num_subcores=NS)
    return pl.kernel(
        _tec_kernel,
        out_type=jax.ShapeDtypeStruct((B, D), jnp.float32),
        mesh=mesh,
        scratch_types=[
            pltpu.VMEM((T, L), jnp.float32),
            pltpu.VMEM((256, L), jnp.int32),
            pltpu.VMEM((L,), jnp.int32),
            pltpu.VMEM((L,), jnp.float32),
        ],
        compiler_params=pltpu.CompilerParams(use_tc_tiling_on_sc=False,
                                             needs_layout_passes=False),
    )(x, lengths_i32)


def kernel(padded_input, lengths):
    med = _sc_median(padded_input, lengths.astype(jnp.int32))
    return jnp.concatenate(
        [med, lengths.astype(jnp.float32)[:, None]], axis=-1)


# speedup vs baseline: 6.9652x; 1.2352x over previous
import numpy as np

import jax
import jax.numpy as jnp
from jax import lax
from jax.experimental import pallas as pl
from jax.experimental.pallas import tpu as pltpu
from jax.experimental.pallas import tpu_sc as plsc

B, T, D = 16, 4096, 128
NC, NS, L = 2, 16, 16
NW = NC * NS
DCH = D // L
NG = B * DCH
GPW = NG // NW
CH = 512

_SIGN = np.uint32(0x80000000)
_PINF = np.float32(np.inf)


def _tec_kernel(x_hbm, len_hbm, out_hbm, buf, hist, lenv, outv):
    wid = lax.axis_index("s") * NC + lax.axis_index("c")
    pltpu.sync_copy(len_hbm, lenv)
    lanes = lax.iota(jnp.int32, L)
    ones = jnp.ones((L,), jnp.int32)
    inf_row = jnp.full((L,), _PINF, jnp.float32)
    lens = lenv[...]

    def zero_hist(j, c):
        hist[j] = jnp.zeros((L,), jnp.int32)
        return c
    lax.fori_loop(0, 256, zero_hist, 0)

    for i in range(GPW):
        g = wid + NW * i
        b = g // DCH
        dc = g % DCH
        l = jnp.sum(jnp.where(lanes == b, lens, 0))
        k0 = (l - 1) // 2

        def copy_chunk(c, carry, b=b, dc=dc):
            pltpu.sync_copy(
                x_hbm.at[b, pl.ds(c * CH, CH), pl.ds(dc * L, L)],
                buf.at[pl.ds(c * CH, CH)])
            return carry
        lax.fori_loop(0, (l + CH - 1) // CH, copy_chunk, 0)
        buf[l] = inf_row
        buf[l + 1] = inf_row
        buf[l + 2] = inf_row

        nt4 = (l + 3) // 4

        def pass0(t, pfx):
            t4 = t * 4
            for j in range(4):
                xi = plsc.bitcast(buf[t4 + j], jnp.uint32)
                key = jnp.where(xi >> 31 == 1, ~xi, xi ^ _SIGN)
                buf[t4 + j] = plsc.bitcast(key, jnp.float32)
                plsc.addupdate_scatter(
                    hist, [(key >> 24).astype(jnp.int32), lanes], ones)
            return pfx
        lax.fori_loop(0, nt4, pass0, jnp.zeros((L,), jnp.uint32))

        prefix = jnp.zeros((L,), jnp.uint32)
        kvec = jnp.broadcast_to(k0, (L,))
        for p in range(4):
            if p > 0:
                hi_sh, lo_sh = 32 - 8 * p, 24 - 8 * p

                def bodyp(t, pfx, hi_sh=hi_sh, lo_sh=lo_sh):
                    t4 = t * 4
                    for j in range(4):
                        key = plsc.bitcast(buf[t4 + j], jnp.uint32)
                        match = (key >> hi_sh) == pfx
                        bkt = ((key >> lo_sh) & 255).astype(jnp.int32)
                        plsc.addupdate_scatter(hist, [bkt, lanes],
                                               jnp.where(match, 1, 0))
                    return pfx
                lax.fori_loop(0, nt4, bodyp, prefix)

            def scan_hist(q, c):
                cnt, sel, base, kv = c
                for j in range(4):
                    row = hist[q * 4 + j]
                    hist[q * 4 + j] = jnp.zeros((L,), jnp.int32)
                    cnt = cnt + row
                    below = cnt <= kv
                    sel = sel + jnp.where(below, 1, 0)
                    base = base + jnp.where(below, row, 0)
                return cnt, sel, base, kv
            z = jnp.zeros((L,), jnp.int32)
            _, sel, base, _ = lax.fori_loop(0, 64, scan_hist,
                                            (z, z, z, kvec))
            prefix = (prefix << 8) | sel.astype(jnp.uint32)
            kvec = kvec - base

        bits = jnp.where(prefix >> 31 == 1, prefix ^ _SIGN, ~prefix)
        outv[...] = plsc.bitcast(bits, jnp.float32)
        pltpu.sync_copy(outv, out_hbm.at[b, pl.ds(dc * L, L)])


@jax.jit
def _sc_median(x, lengths_i32):
    mesh = plsc.VectorSubcoreMesh(core_axis_name="c", subcore_axis_name="s",
                                  num_cores=NC, num_subcores=NS)
    return pl.kernel(
        _tec_kernel,
        out_type=jax.ShapeDtypeStruct((B, D), jnp.float32),
        mesh=mesh,
        scratch_types=[
            pltpu.VMEM((T + 4, L), jnp.float32),
            pltpu.VMEM((256, L), jnp.int32),
            pltpu.VMEM((L,), jnp.int32),
            pltpu.VMEM((L,), jnp.float32),
        ],
        compiler_params=pltpu.CompilerParams(use_tc_tiling_on_sc=False,
                                             needs_layout_passes=False),
    )(x, lengths_i32)


def kernel(padded_input, lengths):
    med = _sc_median(padded_input, lengths.astype(jnp.int32))
    return jnp.concatenate(
        [med, lengths.astype(jnp.float32)[:, None]], axis=-1)


# speedup vs baseline: 11.0842x; 1.5914x over previous
import numpy as np

import jax
import jax.numpy as jnp
from jax import lax
from jax.experimental import pallas as pl
from jax.experimental.pallas import tpu as pltpu
from jax.experimental.pallas import tpu_sc as plsc

B, T, D = 16, 4096, 128
NC, NS, L = 2, 16, 16
NW = NC * NS
DCH = D // L
NG = B * DCH
GPW = NG // NW
CH = 512
DUMP = T + 3

_SIGN = np.uint32(0x80000000)
_PINF = np.float32(np.inf)


def _tec_kernel(x_hbm, len_hbm, out_hbm, buf, hist, lenv, outv):
    wid = lax.axis_index("s") * NC + lax.axis_index("c")
    pltpu.sync_copy(len_hbm, lenv)
    lanes = lax.iota(jnp.int32, L)
    ones = jnp.ones((L,), jnp.int32)
    inf_row = jnp.full((L,), _PINF, jnp.float32)
    lens = lenv[...]

    def zero_hist(j, c):
        hist[j] = jnp.zeros((L,), jnp.int32)
        return c
    lax.fori_loop(0, 256, zero_hist, 0)

    for i in range(GPW):
        g = wid + NW * i
        b = g // DCH
        dc = g % DCH
        l = jnp.sum(jnp.where(lanes == b, lens, 0))
        k0 = (l - 1) // 2

        def copy_chunk(c, carry, b=b, dc=dc):
            pltpu.sync_copy(
                x_hbm.at[b, pl.ds(c * CH, CH), pl.ds(dc * L, L)],
                buf.at[pl.ds(c * CH, CH)])
            return carry
        lax.fori_loop(0, (l + CH - 1) // CH, copy_chunk, 0)
        buf[l] = inf_row
        buf[l + 1] = inf_row
        buf[l + 2] = inf_row

        nt4 = (l + 3) // 4

        def pass0(t, pfx):
            t4 = t * 4
            for j in range(4):
                xi = plsc.bitcast(buf[t4 + j], jnp.uint32)
                key = jnp.where(xi >> 31 == 1, ~xi, xi ^ _SIGN)
                buf[t4 + j] = plsc.bitcast(key, jnp.float32)
                plsc.addupdate_scatter(
                    hist, [(key >> 24).astype(jnp.int32), lanes], ones)
            return pfx
        lax.fori_loop(0, nt4, pass0, jnp.zeros((L,), jnp.uint32))

        def scan(kv):
            def scan_hist(q, c):
                cnt, sel, base, kvc = c
                for j in range(4):
                    row = hist[q * 4 + j]
                    hist[q * 4 + j] = jnp.zeros((L,), jnp.int32)
                    cnt = cnt + row
                    below = cnt <= kvc
                    sel = sel + jnp.where(below, 1, 0)
                    base = base + jnp.where(below, row, 0)
                return cnt, sel, base, kvc
            z = jnp.zeros((L,), jnp.int32)
            _, sel, base, _ = lax.fori_loop(0, 64, scan_hist, (z, z, z, kv))
            return sel, base

        kvec = jnp.broadcast_to(k0, (L,))
        sel, base = scan(kvec)
        prefix = sel.astype(jnp.uint32)
        kvec = kvec - base

        def pass1(t, c):
            pfx, cnt = c
            t4 = t * 4
            for j in range(4):
                key = plsc.bitcast(buf[t4 + j], jnp.uint32)
                match = (key >> 24) == pfx
                inc = jnp.where(match, 1, 0)
                bkt = ((key >> 16) & 255).astype(jnp.int32)
                plsc.addupdate_scatter(hist, [bkt, lanes], inc)
                addr = jnp.where(match, cnt, DUMP)
                plsc.store_scatter(buf, [addr, lanes],
                                   plsc.bitcast(key, jnp.float32))
                cnt = cnt + inc
            return pfx, cnt
        _, cnt = lax.fori_loop(0, nt4, pass1,
                               (prefix, jnp.zeros((L,), jnp.int32)))
        sel, base = scan(kvec)
        prefix = (prefix << 8) | sel.astype(jnp.uint32)
        kvec = kvec - base
        n2 = jnp.max(cnt)

        def pass2(t, c):
            pfx, cv, cnt2 = c
            t4 = t * 4
            for j in range(4):
                key = plsc.bitcast(buf[t4 + j], jnp.uint32)
                match = jnp.logical_and((key >> 16) == pfx,
                                        jnp.broadcast_to(t4 + j, (L,)) < cv)
                inc = jnp.where(match, 1, 0)
                bkt = ((key >> 8) & 255).astype(jnp.int32)
                plsc.addupdate_scatter(hist, [bkt, lanes], inc)
                addr = jnp.where(match, cnt2, DUMP)
                plsc.store_scatter(buf, [addr, lanes],
                                   plsc.bitcast(key, jnp.float32))
                cnt2 = cnt2 + inc
            return pfx, cv, cnt2
        _, _, cnt2 = lax.fori_loop(0, (n2 + 3) // 4, pass2,
                                   (prefix, cnt, jnp.zeros((L,), jnp.int32)))
        sel, base = scan(kvec)
        prefix = (prefix << 8) | sel.astype(jnp.uint32)
        kvec = kvec - base
        n3 = jnp.max(cnt2)

        def pass3(t, c):
            pfx, cv = c
            t4 = t * 4
            for j in range(4):
                key = plsc.bitcast(buf[t4 + j], jnp.uint32)
                match = jnp.logical_and((key >> 8) == pfx,
                                        jnp.broadcast_to(t4 + j, (L,)) < cv)
                bkt = (key & 255).astype(jnp.int32)
                plsc.addupdate_scatter(hist, [bkt, lanes],
                                       jnp.where(match, 1, 0))
            return pfx, cv
        lax.fori_loop(0, (n3 + 3) // 4, pass3, (prefix, cnt2))
        sel, base = scan(kvec)
        prefix = (prefix << 8) | sel.astype(jnp.uint32)

        bits = jnp.where(prefix >> 31 == 1, prefix ^ _SIGN, ~prefix)
        outv[...] = plsc.bitcast(bits, jnp.float32)
        pltpu.sync_copy(outv, out_hbm.at[b, pl.ds(dc * L, L)])


@jax.jit
def _sc_median(x, lengths_i32):
    mesh = plsc.VectorSubcoreMesh(core_axis_name="c", subcore_axis_name="s",
                                  num_cores=NC, num_subcores=NS)
    return pl.kernel(
        _tec_kernel,
        out_type=jax.ShapeDtypeStruct((B, D), jnp.float32),
        mesh=mesh,
        scratch_types=[
            pltpu.VMEM((T + 4, L), jnp.float32),
            pltpu.VMEM((256, L), jnp.int32),
            pltpu.VMEM((L,), jnp.int32),
            pltpu.VMEM((L,), jnp.float32),
        ],
        compiler_params=pltpu.CompilerParams(use_tc_tiling_on_sc=False,
                                             needs_layout_passes=False),
    )(x, lengths_i32)


def kernel(padded_input, lengths):
    med = _sc_median(padded_input, lengths.astype(jnp.int32))
    return jnp.concatenate(
        [med, lengths.astype(jnp.float32)[:, None]], axis=-1)


# speedup vs baseline: 19.7459x; 1.7814x over previous
import numpy as np

import jax
import jax.numpy as jnp
from jax import lax
from jax.experimental import pallas as pl
from jax.experimental.pallas import tpu as pltpu
from jax.experimental.pallas import tpu_sc as plsc

B, T, D = 16, 4096, 128
NC, NS, L = 2, 16, 16
NW = NC * NS
DCH = D // L
NG = B * DCH
GPW = NG // NW
CH = 512
DUMP = T + 3
U = 4

_SIGN = np.uint32(0x80000000)
_PINF = np.float32(np.inf)


def _key(xi):
    return jnp.where(xi >> 31 == 1, ~xi, xi ^ _SIGN)


def _tec_kernel(x_hbm, len_hbm, out_hbm, buf, hist, lenv, outv):
    wid = lax.axis_index("s") * NC + lax.axis_index("c")
    pltpu.sync_copy(len_hbm, lenv)
    lanes = lax.iota(jnp.int32, L)
    ones = jnp.ones((L,), jnp.int32)
    inf_row = jnp.full((L,), _PINF, jnp.float32)
    lens = lenv[...]

    def zero_hist(j, c):
        hist[j] = jnp.zeros((L,), jnp.int32)
        return c
    lax.fori_loop(0, 256, zero_hist, 0)

    for i in range(GPW):
        g = wid + NW * i
        b = g // DCH
        dc = g % DCH
        l = jnp.sum(jnp.where(lanes == b, lens, 0))
        k0 = (l - 1) // 2

        def copy_chunk(c, carry, b=b, dc=dc):
            pltpu.sync_copy(
                x_hbm.at[b, pl.ds(c * CH, CH), pl.ds(dc * L, L)],
                buf.at[pl.ds(c * CH, CH)])
            return carry
        lax.fori_loop(0, (l + CH - 1) // CH, copy_chunk, 0)
        buf[l] = inf_row
        buf[l + 1] = inf_row
        buf[l + 2] = inf_row

        ntu = (l + U - 1) // U

        def pass0(t, pfx):
            tu = t * U
            keys = [_key(plsc.bitcast(buf[tu + j], jnp.uint32))
                    for j in range(U)]
            bkts = [(k >> 24).astype(jnp.int32) for k in keys]
            for j in range(U):
                plsc.addupdate_scatter(hist, [bkts[j], lanes], ones)
            return pfx
        lax.fori_loop(0, ntu, pass0, jnp.zeros((L,), jnp.uint32))

        def scan(kv):
            def scan_hist(q, c):
                cnt, sel, base, kvc = c
                rows = [hist[q * 4 + j] for j in range(4)]
                for j in range(4):
                    hist[q * 4 + j] = jnp.zeros((L,), jnp.int32)
                for j in range(4):
                    cnt = cnt + rows[j]
                    below = cnt <= kvc
                    sel = sel + jnp.where(below, 1, 0)
                    base = base + jnp.where(below, rows[j], 0)
                return cnt, sel, base, kvc
            z = jnp.zeros((L,), jnp.int32)
            _, sel, base, _ = lax.fori_loop(0, 64, scan_hist, (z, z, z, kv))
            return sel, base

        kvec = jnp.broadcast_to(k0, (L,))
        sel, base = scan(kvec)
        prefix = sel.astype(jnp.uint32)
        kvec = kvec - base

        def pass1(t, c):
            pfx, cnt = c
            tu = t * U
            keys = [_key(plsc.bitcast(buf[tu + j], jnp.uint32))
                    for j in range(U)]
            incs, bkts, addrs = [], [], []
            for k in keys:
                match = (k >> 24) == pfx
                inc = jnp.where(match, 1, 0)
                incs.append(inc)
                bkts.append(((k >> 16) & 255).astype(jnp.int32))
                addrs.append(jnp.where(match, cnt, DUMP))
                cnt = cnt + inc
            for j in range(U):
                plsc.store_scatter(buf, [addrs[j], lanes],
                                   plsc.bitcast(keys[j], jnp.float32))
                plsc.addupdate_scatter(hist, [bkts[j], lanes], incs[j])
            return pfx, cnt
        _, cnt = lax.fori_loop(0, ntu, pass1,
                               (prefix, jnp.zeros((L,), jnp.int32)))
        sel, base = scan(kvec)
        prefix = (prefix << 8) | sel.astype(jnp.uint32)
        kvec = kvec - base
        n2 = jnp.max(cnt)

        def pass2(t, c):
            pfx, cv, cnt2 = c
            tu = t * U
            keys = [plsc.bitcast(buf[tu + j], jnp.uint32) for j in range(U)]
            incs, bkts, addrs = [], [], []
            for j in range(U):
                k = keys[j]
                match = jnp.logical_and(
                    (k >> 16) == pfx,
                    jnp.broadcast_to(tu + j, (L,)) < cv)
                inc = jnp.where(match, 1, 0)
                incs.append(inc)
                bkts.append(((k >> 8) & 255).astype(jnp.int32))
                addrs.append(jnp.where(match, cnt2, DUMP))
                cnt2 = cnt2 + inc
            for j in range(U):
                plsc.store_scatter(buf, [addrs[j], lanes],
                                   plsc.bitcast(keys[j], jnp.float32))
                plsc.addupdate_scatter(hist, [bkts[j], lanes], incs[j])
            return pfx, cv, cnt2
        _, _, cnt2 = lax.fori_loop(0, (n2 + U - 1) // U, pass2,
                                   (prefix, cnt, jnp.zeros((L,), jnp.int32)))
        sel, base = scan(kvec)
        prefix = (prefix << 8) | sel.astype(jnp.uint32)
        kvec = kvec - base
        n3 = jnp.max(cnt2)

        def pass3(t, c):
            pfx, cv = c
            tu = t * U
            keys = [plsc.bitcast(buf[tu + j], jnp.uint32) for j in range(U)]
            incs, bkts = [], []
            for j in range(U):
                k = keys[j]
                match = jnp.logical_and(
                    (k >> 8) == pfx,
                    jnp.broadcast_to(tu + j, (L,)) < cv)
                incs.append(jnp.where(match, 1, 0))
                bkts.append((k & 255).astype(jnp.int32))
            for j in range(U):
                plsc.addupdate_scatter(hist, [bkts[j], lanes], incs[j])
            return pfx, cv
        lax.fori_loop(0, (n3 + U - 1) // U, pass3, (prefix, cnt2))
        sel, base = scan(kvec)
        prefix = (prefix << 8) | sel.astype(jnp.uint32)

        bits = jnp.where(prefix >> 31 == 1, prefix ^ _SIGN, ~prefix)
        outv[...] = plsc.bitcast(bits, jnp.float32)
        pltpu.sync_copy(outv, out_hbm.at[b, pl.ds(dc * L, L)])


@jax.jit
def _sc_median(x, lengths_i32):
    mesh = plsc.VectorSubcoreMesh(core_axis_name="c", subcore_axis_name="s",
                                  num_cores=NC, num_subcores=NS)
    return pl.kernel(
        _tec_kernel,
        out_type=jax.ShapeDtypeStruct((B, D), jnp.float32),
        mesh=mesh,
        scratch_types=[
            pltpu.VMEM((T + 4, L), jnp.float32),
            pltpu.VMEM((256, L), jnp.int32),
            pltpu.VMEM((L,), jnp.int32),
            pltpu.VMEM((L,), jnp.float32),
        ],
        compiler_params=pltpu.CompilerParams(use_tc_tiling_on_sc=False,
                                             needs_layout_passes=False),
    )(x, lengths_i32)


def kernel(padded_input, lengths):
    med = _sc_median(padded_input, lengths.astype(jnp.int32))
    return jnp.concatenate(
        [med, lengths.astype(jnp.float32)[:, None]], axis=-1)


# speedup vs baseline: 21.3711x; 1.0823x over previous
import numpy as np

import jax
import jax.numpy as jnp
from jax import lax
from jax.experimental import pallas as pl
from jax.experimental.pallas import tpu as pltpu
from jax.experimental.pallas import tpu_sc as plsc

B, T, D = 16, 4096, 128
NC, NS, L = 2, 16, 16
NW = NC * NS
DCH = D // L
NG = B * DCH
GPW = NG // NW
CH = 512
DUMP = T + 3
U = 4

_SIGN = np.uint32(0x80000000)
_PINF = np.float32(np.inf)


def _key(xi):
    return jnp.where(xi >> 31 == 1, ~xi, xi ^ _SIGN)


def _tec_kernel(x_hbm, len_hbm, out_hbm, buf, hist, lenv, outv, sem):
    wid = lax.axis_index("s") * NC + lax.axis_index("c")
    pltpu.sync_copy(len_hbm, lenv)
    lanes = lax.iota(jnp.int32, L)
    ones = jnp.ones((L,), jnp.int32)
    inf_row = jnp.full((L,), _PINF, jnp.float32)
    lens = lenv[...]

    def zero_hist(j, c):
        hist[j] = jnp.zeros((L,), jnp.int32)
        return c
    lax.fori_loop(0, 256, zero_hist, 0)

    for i in range(GPW):
        g = wid + NW * i
        b = g // DCH
        dc = g % DCH
        l = jnp.sum(jnp.where(lanes == b, lens, 0))
        k0 = (l - 1) // 2

        nch = (l + CH - 1) // CH

        def fire_chunk(c, carry, b=b, dc=dc):
            pltpu.async_copy(
                x_hbm.at[b, pl.ds(c * CH, CH), pl.ds(dc * L, L)],
                buf.at[pl.ds(c * CH, CH)], sem)
            return carry
        lax.fori_loop(0, nch, fire_chunk, 0)

        def drain_chunk(c, carry, b=b, dc=dc):
            pltpu.make_async_copy(
                x_hbm.at[b, pl.ds(c * CH, CH), pl.ds(dc * L, L)],
                buf.at[pl.ds(c * CH, CH)], sem).wait()
            return carry
        lax.fori_loop(0, nch, drain_chunk, 0)
        buf[l] = inf_row
        buf[l + 1] = inf_row
        buf[l + 2] = inf_row

        ntu = (l + U - 1) // U

        def pass0(t, pfx):
            tu = t * U
            keys = [_key(plsc.bitcast(buf[tu + j], jnp.uint32))
                    for j in range(U)]
            bkts = [(k >> 24).astype(jnp.int32) for k in keys]
            for j in range(U):
                plsc.addupdate_scatter(hist, [bkts[j], lanes], ones)
            return pfx
        lax.fori_loop(0, ntu, pass0, jnp.zeros((L,), jnp.uint32))

        def scan(kv):
            def scan_hist(q, c):
                cnt, sel, base, kvc = c
                rows = [hist[q * 4 + j] for j in range(4)]
                for j in range(4):
                    hist[q * 4 + j] = jnp.zeros((L,), jnp.int32)
                for j in range(4):
                    cnt = cnt + rows[j]
                    below = cnt <= kvc
                    sel = sel + jnp.where(below, 1, 0)
                    base = base + jnp.where(below, rows[j], 0)
                return cnt, sel, base, kvc
            z = jnp.zeros((L,), jnp.int32)
            _, sel, base, _ = lax.fori_loop(0, 64, scan_hist, (z, z, z, kv))
            return sel, base

        kvec = jnp.broadcast_to(k0, (L,))
        sel, base = scan(kvec)
        prefix = sel.astype(jnp.uint32)
        kvec = kvec - base

        def pass1(t, c):
            pfx, cnt = c
            tu = t * U
            keys = [_key(plsc.bitcast(buf[tu + j], jnp.uint32))
                    for j in range(U)]
            incs, bkts, addrs = [], [], []
            for k in keys:
                match = (k >> 24) == pfx
                inc = jnp.where(match, 1, 0)
                incs.append(inc)
                bkts.append(((k >> 16) & 255).astype(jnp.int32))
                addrs.append(jnp.where(match, cnt, DUMP))
                cnt = cnt + inc
            for j in range(U):
                plsc.store_scatter(buf, [addrs[j], lanes],
                                   plsc.bitcast(keys[j], jnp.float32))
                plsc.addupdate_scatter(hist, [bkts[j], lanes], incs[j])
            return pfx, cnt
        _, cnt = lax.fori_loop(0, ntu, pass1,
                               (prefix, jnp.zeros((L,), jnp.int32)))
        sel, base = scan(kvec)
        prefix = (prefix << 8) | sel.astype(jnp.uint32)
        kvec = kvec - base
        n2 = jnp.max(cnt)

        def pass2(t, c):
            pfx, cv, cnt2 = c
            tu = t * U
            keys = [plsc.bitcast(buf[tu + j], jnp.uint32) for j in range(U)]
            incs, bkts, addrs = [], [], []
            for j in range(U):
                k = keys[j]
                match = jnp.logical_and(
                    (k >> 16) == pfx,
                    jnp.broadcast_to(tu + j, (L,)) < cv)
                inc = jnp.where(match, 1, 0)
                incs.append(inc)
                bkts.append(((k >> 8) & 255).astype(jnp.int32))
                addrs.append(jnp.where(match, cnt2, DUMP))
                cnt2 = cnt2 + inc
            for j in range(U):
                plsc.store_scatter(buf, [addrs[j], lanes],
                                   plsc.bitcast(keys[j], jnp.float32))
                plsc.addupdate_scatter(hist, [bkts[j], lanes], incs[j])
            return pfx, cv, cnt2
        _, _, cnt2 = lax.fori_loop(0, (n2 + U - 1) // U, pass2,
                                   (prefix, cnt, jnp.zeros((L,), jnp.int32)))
        sel, base = scan(kvec)
        prefix = (prefix << 8) | sel.astype(jnp.uint32)
        kvec = kvec - base
        n3 = jnp.max(cnt2)

        def pass3(t, c):
            pfx, cv = c
            tu = t * U
            keys = [plsc.bitcast(buf[tu + j], jnp.uint32) for j in range(U)]
            incs, bkts = [], []
            for j in range(U):
                k = keys[j]
                match = jnp.logical_and(
                    (k >> 8) == pfx,
                    jnp.broadcast_to(tu + j, (L,)) < cv)
                incs.append(jnp.where(match, 1, 0))
                bkts.append((k & 255).astype(jnp.int32))
            for j in range(U):
                plsc.addupdate_scatter(hist, [bkts[j], lanes], incs[j])
            return pfx, cv
        lax.fori_loop(0, (n3 + U - 1) // U, pass3, (prefix, cnt2))
        sel, base = scan(kvec)
        prefix = (prefix << 8) | sel.astype(jnp.uint32)

        bits = jnp.where(prefix >> 31 == 1, prefix ^ _SIGN, ~prefix)
        outv[...] = plsc.bitcast(bits, jnp.float32)
        pltpu.sync_copy(outv, out_hbm.at[b, pl.ds(dc * L, L)])


@jax.jit
def _sc_median(x, lengths_i32):
    mesh = plsc.VectorSubcoreMesh(core_axis_name="c", subcore_axis_name="s",
                                  num_cores=NC, num_subcores=NS)
    return pl.kernel(
        _tec_kernel,
        out_type=jax.ShapeDtypeStruct((B, D), jnp.float32),
        mesh=mesh,
        scratch_types=[
            pltpu.VMEM((T + 4, L), jnp.float32),
            pltpu.VMEM((256, L), jnp.int32),
            pltpu.VMEM((L,), jnp.int32),
            pltpu.VMEM((L,), jnp.float32),
            pltpu.SemaphoreType.DMA,
        ],
        compiler_params=pltpu.CompilerParams(use_tc_tiling_on_sc=False,
                                             needs_layout_passes=False),
    )(x, lengths_i32)


def kernel(padded_input, lengths):
    med = _sc_median(padded_input, lengths.astype(jnp.int32))
    return jnp.concatenate(
        [med, lengths.astype(jnp.float32)[:, None]], axis=-1)
